# baseline (speedup 1.0000x reference)
import functools

import jax
import jax.numpy as jnp
from jax import lax
from jax.experimental import pallas as pl
from jax.experimental.pallas import tpu as pltpu
from jax.experimental.pallas import tpu_sc as plsc

_NC = 2
_NS = 16
_NW = _NC * _NS
_PER_W = 256
_PAD_M = _NW * _PER_W
_LANES = 16

_SC_ROWS = 32768
_CHUNK = 16384
_UNROLL = 8

_C0 = 1.4418798685073853
_C1 = -0.708865225315094
_C2 = 0.41524556279182434
_C3 = -0.19351652264595032
_C4 = 0.04526829347014427
_LN2 = 0.6931471805599453


def _focal_term_log2(o, t):
    x = 1.0 - o
    bits = lax.bitcast_convert_type(x, jnp.int32)
    e = lax.shift_right_logical(bits, 23)
    ef = e.astype(jnp.float32) - 127.0
    mbits = (bits & jnp.int32(0x007FFFFF)) | jnp.int32(0x3F800000)
    m = lax.bitcast_convert_type(mbits, jnp.float32)
    tm = m - 1.0
    p = jnp.float32(_C4)
    p = p * tm + jnp.float32(_C3)
    p = p * tm + jnp.float32(_C2)
    p = p * tm + jnp.float32(_C1)
    p = p * tm + jnp.float32(_C0)
    log2x = ef + tm * p
    u = 1.0 - t
    u2 = u * u
    return log2x * (o * o) * (u2 * u2)


def _sc_kernel(n_total, chw, hw, bm, out_flat, tgt_flat, ind_pad, cat_pad):
    sc_elems = _SC_ROWS * 128
    per_w = sc_elems // _NW
    n_chunks = per_w // _CHUNK
    groups = _CHUNK // (_LANES * _UNROLL)

    @functools.partial(
        pl.kernel,
        out_type=(jax.ShapeDtypeStruct((_PAD_M,), jnp.float32),
                  jax.ShapeDtypeStruct((_NW, _LANES), jnp.float32)),
        mesh=plsc.VectorSubcoreMesh(core_axis_name="c", subcore_axis_name="s"),
        scratch_types=[
            pltpu.VMEM((_PER_W,), jnp.int32),
            pltpu.VMEM((_PER_W,), jnp.int32),
            pltpu.VMEM((2, 128), jnp.int32),
            pltpu.VMEM((2, 128), jnp.float32),
            pltpu.VMEM((2, _CHUNK), jnp.float32),
            pltpu.VMEM((2, _CHUNK), jnp.float32),
            pltpu.VMEM((_LANES,), jnp.float32),
            pltpu.SemaphoreType.DMA,
            pltpu.SemaphoreType.DMA,
            pltpu.SemaphoreType.DMA,
            pltpu.SemaphoreType.DMA,
        ],
    )
    def k(out_hbm, tgt_hbm, ind_hbm, cat_hbm, pos_hbm, part_hbm,
          ind_v, cat_v, idx_v, vals_v, obuf, tbuf, accv,
          sem_o0, sem_o1, sem_t0, sem_t1):
        wid = lax.axis_index("s") * _NC + lax.axis_index("c")

        gbase = wid * _PER_W
        pltpu.sync_copy(ind_hbm.at[pl.ds(gbase, _PER_W)], ind_v)
        pltpu.sync_copy(cat_hbm.at[pl.ds(gbase, _PER_W)], cat_v)
        lane = lax.iota(jnp.int32, _LANES)
        for k16 in range(_PER_W // _LANES):
            j = gbase + k16 * _LANES + lane
            b = lax.div(j, bm)
            fi = (b * chw + cat_v[pl.ds(k16 * _LANES, _LANES)] * hw
                  + ind_v[pl.ds(k16 * _LANES, _LANES)])
            row, col = divmod(k16 * _LANES, 128)
            idx_v[row, pl.ds(col, _LANES)] = fi
        for r in range(2):
            pltpu.async_copy(out_hbm.at[idx_v.at[r]], vals_v.at[r],
                             sem_o0).wait()
            pltpu.sync_copy(vals_v.at[r],
                            pos_hbm.at[pl.ds(gbase + r * 128, 128)])

        wbase = wid * per_w
        osems = (sem_o0, sem_o1)
        tsems = (sem_t0, sem_t1)

        def issue(chunk, buf):
            off = wbase + chunk * _CHUNK
            co = pltpu.async_copy(out_hbm.at[pl.ds(off, _CHUNK)],
                                  obuf.at[buf], osems[buf])
            ct = pltpu.async_copy(tgt_hbm.at[pl.ds(off, _CHUNK)],
                                  tbuf.at[buf], tsems[buf])
            return co, ct

        accv[...] = jnp.zeros((_LANES,), jnp.float32)
        pending = issue(0, 0)
        for chunk in range(n_chunks):
            buf = chunk % 2
            pending[0].wait()
            pending[1].wait()
            if chunk + 1 < n_chunks:
                pending = issue(chunk + 1, 1 - buf)

            def body(g, acc):
                base = g * (_LANES * _UNROLL)
                for u in range(_UNROLL):
                    off = base + u * _LANES
                    o = obuf[buf, pl.ds(off, _LANES)]
                    t = tbuf[buf, pl.ds(off, _LANES)]
                    acc = acc + _focal_term_log2(o, t)
                return acc

            accv[...] = lax.fori_loop(0, groups, body, accv[...])
        pltpu.sync_copy(accv, part_hbm.at[wid])

    return k(out_flat, tgt_flat, ind_pad, cat_pad)


def _tc_dense(out_ref, tgt_ref, part_ref):
    o = out_ref[...]
    t = tgt_ref[...]
    t1 = 1.0 - t
    t2 = t1 * t1
    gt = t2 * t2
    term = jnp.log(1.0 - o) * (o * o) * gt
    part_ref[...] = jnp.sum(term, axis=0, keepdims=True)[None]


def _tc_combine(part_ref, scp_ref, pos_ref, mk_ref, res_ref):
    neg = jnp.sum(part_ref[...]) + jnp.sum(scp_ref[...]) * jnp.float32(_LN2)
    pv = pos_ref[...]
    mk = mk_ref[...]
    pt = jnp.where(mk != 0.0, jnp.log(pv) * (1.0 - pv) * (1.0 - pv) * mk, 0.0)
    pos_loss = jnp.sum(pt)
    num_pos = jnp.sum(mk)
    val = jnp.where(num_pos == 0.0, -neg, -(pos_loss + neg) / num_pos)
    res_ref[...] = jnp.full((1, 1), val, jnp.float32)


def kernel(out, target, ind, mask, cat):
    B, C, H, W = out.shape
    M = ind.shape[1]
    n_total = B * C * H * W
    chw = C * H * W
    hw = H * W

    pad = _PAD_M - B * M
    ind_pad = jnp.concatenate(
        [ind.reshape(-1).astype(jnp.int32),
         jnp.full((pad,), -chw, jnp.int32)])
    cat_pad = jnp.concatenate(
        [cat.reshape(-1).astype(jnp.int32), jnp.zeros((pad,), jnp.int32)])
    mask_pad = jnp.concatenate(
        [mask.reshape(-1).astype(jnp.float32), jnp.zeros((pad,), jnp.float32)])

    pos_vals, sc_parts = _sc_kernel(n_total, chw, hw, M,
                                    out.reshape(n_total),
                                    target.reshape(n_total),
                                    ind_pad, cat_pad)

    rows = n_total // 128
    blk = 16384
    skip = _SC_ROWS // blk
    grid = rows // blk - skip
    out2 = out.reshape(rows, 128)
    tgt2 = target.reshape(rows, 128)
    pos2 = pos_vals.reshape(_PAD_M // 128, 128)
    mk2 = mask_pad.reshape(_PAD_M // 128, 128)
    scp2 = sc_parts.reshape(_NW * _LANES // 128, 128)

    partials = pl.pallas_call(
        _tc_dense,
        grid=(grid,),
        in_specs=[
            pl.BlockSpec((blk, 128), lambda i: (i + skip, 0)),
            pl.BlockSpec((blk, 128), lambda i: (i + skip, 0)),
        ],
        out_specs=pl.BlockSpec((1, 1, 128), lambda i: (i, 0, 0)),
        out_shape=jax.ShapeDtypeStruct((grid, 1, 128), jnp.float32),
        compiler_params=pltpu.CompilerParams(
            dimension_semantics=("arbitrary",)),
    )(out2, tgt2)
    partials = partials.reshape(grid, 128)

    res = pl.pallas_call(
        _tc_combine,
        in_specs=[
            pl.BlockSpec((grid, 128), lambda: (0, 0)),
            pl.BlockSpec(scp2.shape, lambda: (0, 0)),
            pl.BlockSpec((_PAD_M // 128, 128), lambda: (0, 0)),
            pl.BlockSpec((_PAD_M // 128, 128), lambda: (0, 0)),
        ],
        out_specs=pl.BlockSpec((1, 1), lambda: (0, 0)),
        out_shape=jax.ShapeDtypeStruct((1, 1), jnp.float32),
    )(partials, scp2, pos2, mk2)

    return res.reshape(())


# speedup vs baseline: 1.0713x; 1.0713x over previous
import functools

import jax
import jax.numpy as jnp
from jax import lax
from jax.experimental import pallas as pl
from jax.experimental.pallas import tpu as pltpu
from jax.experimental.pallas import tpu_sc as plsc

_NC = 2
_NS = 16
_NW = _NC * _NS
_PER_W = 256
_PAD_M = _NW * _PER_W
_LANES = 16

_C0 = 1.4418798685073853
_C1 = -0.708865225315094
_C2 = 0.41524556279182434
_C3 = -0.19351652264595032
_C4 = 0.04526829347014427
_LN2 = 0.6931471805599453


def _softlog(x):
    bits = lax.bitcast_convert_type(x, jnp.int32)
    e = lax.shift_right_logical(bits, 23)
    ef = e.astype(jnp.float32) - 127.0
    mbits = (bits & jnp.int32(0x007FFFFF)) | jnp.int32(0x3F800000)
    m = lax.bitcast_convert_type(mbits, jnp.float32)
    tm = m - 1.0
    p = jnp.float32(_C4)
    p = p * tm + jnp.float32(_C3)
    p = p * tm + jnp.float32(_C2)
    p = p * tm + jnp.float32(_C1)
    p = p * tm + jnp.float32(_C0)
    lg = (ef + tm * p) * jnp.float32(_LN2)
    return jnp.where(x == 0.0, jnp.float32(-jnp.inf), lg)


def _sc_pos(n_total, chw, hw, bm, out_flat, ind_pad, cat_pad, mask_pad):
    @functools.partial(
        pl.kernel,
        out_type=jax.ShapeDtypeStruct((_NW, 2, _LANES), jnp.float32),
        mesh=plsc.VectorSubcoreMesh(core_axis_name="c", subcore_axis_name="s"),
        scratch_types=[
            pltpu.VMEM((_PER_W,), jnp.int32),
            pltpu.VMEM((_PER_W,), jnp.int32),
            pltpu.VMEM((_PER_W,), jnp.float32),
            pltpu.VMEM((2, 128), jnp.int32),
            pltpu.VMEM((2, 128), jnp.float32),
            pltpu.VMEM((2, _LANES), jnp.float32),
            pltpu.SemaphoreType.DMA,
        ],
    )
    def k(out_hbm, ind_hbm, cat_hbm, mask_hbm, part_hbm,
          ind_v, cat_v, mask_v, idx_v, vals_v, pacc, sem):
        wid = lax.axis_index("s") * _NC + lax.axis_index("c")
        base = wid * _PER_W
        pltpu.sync_copy(ind_hbm.at[pl.ds(base, _PER_W)], ind_v)
        pltpu.sync_copy(cat_hbm.at[pl.ds(base, _PER_W)], cat_v)
        pltpu.sync_copy(mask_hbm.at[pl.ds(base, _PER_W)], mask_v)
        lane = lax.iota(jnp.int32, _LANES)
        for k16 in range(_PER_W // _LANES):
            j = base + k16 * _LANES + lane
            b = lax.div(j, bm)
            fi = (b * chw + cat_v[pl.ds(k16 * _LANES, _LANES)] * hw
                  + ind_v[pl.ds(k16 * _LANES, _LANES)])
            row, col = divmod(k16 * _LANES, 128)
            idx_v[row, pl.ds(col, _LANES)] = fi
        for r in range(2):
            pltpu.async_copy(out_hbm.at[idx_v.at[r]], vals_v.at[r],
                             sem).wait()

        pos_acc = jnp.zeros((_LANES,), jnp.float32)
        msk_acc = jnp.zeros((_LANES,), jnp.float32)
        for k16 in range(_PER_W // _LANES):
            row, col = divmod(k16 * _LANES, 128)
            pv = vals_v[row, pl.ds(col, _LANES)]
            mk = mask_v[pl.ds(k16 * _LANES, _LANES)]
            q = 1.0 - pv
            pt = jnp.where(mk != 0.0, _softlog(pv) * q * q * mk, 0.0)
            pos_acc = pos_acc + pt
            msk_acc = msk_acc + mk
        pacc[0, :] = pos_acc
        pacc[1, :] = msk_acc
        pltpu.sync_copy(pacc, part_hbm.at[wid])

    return k(out_flat, ind_pad, cat_pad, mask_pad)


def _tc_dense(out_ref, tgt_ref, res_ref, acc_ref):
    i = pl.program_id(0)
    o = out_ref[...]
    t = tgt_ref[...]
    t1 = 1.0 - t
    t2 = t1 * t1
    gt = t2 * t2
    part = jnp.sum(jnp.log(1.0 - o) * (o * o) * gt)

    @pl.when(i == 0)
    def _init():
        acc_ref[0] = 0.0

    acc_ref[0] += part

    @pl.when(i == pl.num_programs(0) - 1)
    def _fin():
        res_ref[...] = jnp.full((1, 1), acc_ref[0], jnp.float32)


def kernel(out, target, ind, mask, cat):
    B, C, H, W = out.shape
    M = ind.shape[1]
    n_total = B * C * H * W
    chw = C * H * W
    hw = H * W

    pad = _PAD_M - B * M
    ind_pad = jnp.concatenate(
        [ind.reshape(-1).astype(jnp.int32),
         jnp.full((pad,), -chw, jnp.int32)])
    cat_pad = jnp.concatenate(
        [cat.reshape(-1).astype(jnp.int32), jnp.zeros((pad,), jnp.int32)])
    mask_pad = jnp.concatenate(
        [mask.reshape(-1).astype(jnp.float32), jnp.zeros((pad,), jnp.float32)])

    parts = _sc_pos(n_total, chw, hw, M, out.reshape(n_total),
                    ind_pad, cat_pad, mask_pad)

    rows = n_total // 128
    blk = 16384
    grid = rows // blk
    out2 = out.reshape(rows, 128)
    tgt2 = target.reshape(rows, 128)

    neg_arr = pl.pallas_call(
        _tc_dense,
        grid=(grid,),
        in_specs=[
            pl.BlockSpec((blk, 128), lambda i: (i, 0)),
            pl.BlockSpec((blk, 128), lambda i: (i, 0)),
        ],
        out_specs=pl.BlockSpec((1, 1), lambda i: (0, 0)),
        out_shape=jax.ShapeDtypeStruct((1, 1), jnp.float32),
        scratch_shapes=[pltpu.SMEM((1,), jnp.float32)],
        compiler_params=pltpu.CompilerParams(
            dimension_semantics=("arbitrary",)),
    )(out2, tgt2)

    neg = neg_arr.reshape(())
    pos_loss = jnp.sum(parts[:, 0, :])
    num_pos = jnp.sum(parts[:, 1, :])
    return jnp.where(num_pos == 0.0, -neg, -(pos_loss + neg) / num_pos)


# speedup vs baseline: 1.0764x; 1.0047x over previous
import functools

import jax
import jax.numpy as jnp
from jax import lax
from jax.experimental import pallas as pl
from jax.experimental.pallas import tpu as pltpu
from jax.experimental.pallas import tpu_sc as plsc

_NC = 2
_NS = 16
_NW = _NC * _NS
_PER_W = 256
_PAD_M = _NW * _PER_W
_LANES = 16


def _sc_gather(n_total, chw, hw, bm, out_flat, ind_pad, cat_pad):
    @functools.partial(
        pl.kernel,
        out_type=jax.ShapeDtypeStruct((_PAD_M,), jnp.float32),
        mesh=plsc.VectorSubcoreMesh(core_axis_name="c", subcore_axis_name="s"),
        scratch_types=[
            pltpu.VMEM((_PER_W,), jnp.int32),
            pltpu.VMEM((_PER_W,), jnp.int32),
            pltpu.VMEM((2, 128), jnp.int32),
            pltpu.VMEM((2, 128), jnp.float32),
            pltpu.SemaphoreType.DMA,
        ],
    )
    def k(out_hbm, ind_hbm, cat_hbm, pos_hbm, ind_v, cat_v, idx_v, vals_v, sem):
        wid = lax.axis_index("s") * _NC + lax.axis_index("c")
        base = wid * _PER_W
        pltpu.sync_copy(ind_hbm.at[pl.ds(base, _PER_W)], ind_v)
        pltpu.sync_copy(cat_hbm.at[pl.ds(base, _PER_W)], cat_v)
        lane = lax.iota(jnp.int32, _LANES)
        for k16 in range(_PER_W // _LANES):
            j = base + k16 * _LANES + lane
            b = lax.div(j, bm)
            fi = (b * chw + cat_v[pl.ds(k16 * _LANES, _LANES)] * hw
                  + ind_v[pl.ds(k16 * _LANES, _LANES)])
            row, col = divmod(k16 * _LANES, 128)
            idx_v[row, pl.ds(col, _LANES)] = fi
        for r in range(2):
            pltpu.async_copy(out_hbm.at[idx_v.at[r]], vals_v.at[r], sem).wait()
            pltpu.sync_copy(vals_v.at[r],
                            pos_hbm.at[pl.ds(base + r * 128, 128)])

    return k(out_flat, ind_pad, cat_pad)


def _tc_body(out_ref, tgt_ref, pos_ref, mk_ref, res_ref, acc_ref):
    i = pl.program_id(0)
    o = out_ref[...]
    t = tgt_ref[...]
    t1 = 1.0 - t
    t2 = t1 * t1
    gt = t2 * t2
    part = jnp.sum(jnp.log(1.0 - o) * (o * o) * gt)

    @pl.when(i == 0)
    def _init():
        acc_ref[0] = 0.0

    acc_ref[0] += part

    @pl.when(i == pl.num_programs(0) - 1)
    def _fin():
        pv = pos_ref[...]
        mk = mk_ref[...]
        pt = jnp.where(mk != 0.0,
                       jnp.log(pv) * (1.0 - pv) * (1.0 - pv) * mk, 0.0)
        pos_loss = jnp.sum(pt)
        num_pos = jnp.sum(mk)
        neg = acc_ref[0]
        val = jnp.where(num_pos == 0.0, -neg, -(pos_loss + neg) / num_pos)
        res_ref[...] = jnp.full((1, 1), val, jnp.float32)


def kernel(out, target, ind, mask, cat):
    B, C, H, W = out.shape
    M = ind.shape[1]
    n_total = B * C * H * W
    chw = C * H * W
    hw = H * W

    pad = _PAD_M - B * M
    ind_pad = jnp.concatenate(
        [ind.reshape(-1).astype(jnp.int32),
         jnp.full((pad,), -chw, jnp.int32)])
    cat_pad = jnp.concatenate(
        [cat.reshape(-1).astype(jnp.int32), jnp.zeros((pad,), jnp.int32)])
    mask_pad = jnp.concatenate(
        [mask.reshape(-1).astype(jnp.float32), jnp.zeros((pad,), jnp.float32)])

    pos_vals = _sc_gather(n_total, chw, hw, M, out.reshape(n_total),
                          ind_pad, cat_pad)

    rows = n_total // 128
    blk = 20480
    grid = rows // blk
    out2 = out.reshape(rows, 128)
    tgt2 = target.reshape(rows, 128)
    pos2 = pos_vals.reshape(_PAD_M // 128, 128)
    mk2 = mask_pad.reshape(_PAD_M // 128, 128)

    res = pl.pallas_call(
        _tc_body,
        grid=(grid,),
        in_specs=[
            pl.BlockSpec((blk, 128), lambda i: (i, 0)),
            pl.BlockSpec((blk, 128), lambda i: (i, 0)),
            pl.BlockSpec((_PAD_M // 128, 128), lambda i: (0, 0)),
            pl.BlockSpec((_PAD_M // 128, 128), lambda i: (0, 0)),
        ],
        out_specs=pl.BlockSpec((1, 1), lambda i: (0, 0)),
        out_shape=jax.ShapeDtypeStruct((1, 1), jnp.float32),
        scratch_shapes=[pltpu.SMEM((1,), jnp.float32)],
        compiler_params=pltpu.CompilerParams(
            dimension_semantics=("arbitrary",)),
    )(out2, tgt2, pos2, mk2)

    return res.reshape(())


# speedup vs baseline: 1.0927x; 1.0152x over previous
import functools

import jax
import jax.numpy as jnp
from jax import lax
from jax.experimental import pallas as pl
from jax.experimental.pallas import tpu as pltpu
from jax.experimental.pallas import tpu_sc as plsc

_NC = 2
_NS = 16
_NW = _NC * _NS
_PER_W = 256
_PAD_M = _NW * _PER_W
_LANES = 16


def _sc_gather(n_total, chw, hw, bm, out_flat, ind_pad, cat_pad):
    @functools.partial(
        pl.kernel,
        out_type=jax.ShapeDtypeStruct((_PAD_M,), jnp.float32),
        mesh=plsc.VectorSubcoreMesh(core_axis_name="c", subcore_axis_name="s"),
        scratch_types=[
            pltpu.VMEM((_PER_W,), jnp.int32),
            pltpu.VMEM((_PER_W,), jnp.int32),
            pltpu.VMEM((2, 128), jnp.int32),
            pltpu.VMEM((2, 128), jnp.float32),
            pltpu.SemaphoreType.DMA,
        ],
    )
    def k(out_hbm, ind_hbm, cat_hbm, pos_hbm, ind_v, cat_v, idx_v, vals_v, sem):
        wid = lax.axis_index("s") * _NC + lax.axis_index("c")
        base = wid * _PER_W
        pltpu.sync_copy(ind_hbm.at[pl.ds(base, _PER_W)], ind_v)
        pltpu.sync_copy(cat_hbm.at[pl.ds(base, _PER_W)], cat_v)
        lane = lax.iota(jnp.int32, _LANES)
        for k16 in range(_PER_W // _LANES):
            j = base + k16 * _LANES + lane
            b = lax.div(j, bm)
            fi = (b * chw + cat_v[pl.ds(k16 * _LANES, _LANES)] * hw
                  + ind_v[pl.ds(k16 * _LANES, _LANES)])
            row, col = divmod(k16 * _LANES, 128)
            idx_v[row, pl.ds(col, _LANES)] = fi
        for r in range(2):
            pltpu.async_copy(out_hbm.at[idx_v.at[r]], vals_v.at[r], sem).wait()
            pltpu.sync_copy(vals_v.at[r],
                            pos_hbm.at[pl.ds(base + r * 128, 128)])

    return k(out_flat, ind_pad, cat_pad)


def _tc_body(out_ref, tgt_ref, pos_ref, mk_ref, res_ref, acc_ref):
    i = pl.program_id(0)
    o = out_ref[...]
    t = tgt_ref[...]
    t1 = 1.0 - t
    t2 = t1 * t1
    gt = t2 * t2
    part = jnp.sum(jnp.log(1.0 - o) * (o * o) * gt)

    @pl.when(i == 0)
    def _init():
        acc_ref[0] = 0.0

    acc_ref[0] += part

    @pl.when(i == pl.num_programs(0) - 1)
    def _fin():
        pv = pos_ref[...]
        mk = mk_ref[...]
        pt = jnp.where(mk != 0.0,
                       jnp.log(pv) * (1.0 - pv) * (1.0 - pv) * mk, 0.0)
        pos_loss = jnp.sum(pt)
        num_pos = jnp.sum(mk)
        neg = acc_ref[0]
        val = jnp.where(num_pos == 0.0, -neg, -(pos_loss + neg) / num_pos)
        res_ref[...] = jnp.full((1, 1), val, jnp.float32)


def kernel(out, target, ind, mask, cat):
    B, C, H, W = out.shape
    M = ind.shape[1]
    n_total = B * C * H * W
    chw = C * H * W
    hw = H * W

    pad = _PAD_M - B * M
    ind_pad = jnp.concatenate(
        [ind.reshape(-1).astype(jnp.int32),
         jnp.full((pad,), -chw, jnp.int32)])
    cat_pad = jnp.concatenate(
        [cat.reshape(-1).astype(jnp.int32), jnp.zeros((pad,), jnp.int32)])
    mask_pad = jnp.concatenate(
        [mask.reshape(-1).astype(jnp.float32), jnp.zeros((pad,), jnp.float32)])

    pos_vals = _sc_gather(n_total, chw, hw, M, out.reshape(n_total),
                          ind_pad, cat_pad)

    rows = n_total // 128
    blk = 16384
    grid = rows // blk
    out2 = out.reshape(rows, 128)
    tgt2 = target.reshape(rows, 128)
    pos2 = pos_vals.reshape(_PAD_M // 128, 128)
    mk2 = mask_pad.reshape(_PAD_M // 128, 128)

    res = pl.pallas_call(
        _tc_body,
        grid=(grid,),
        in_specs=[
            pl.BlockSpec((blk, 128), lambda i: (i, 0)),
            pl.BlockSpec((blk, 128), lambda i: (i, 0)),
            pl.BlockSpec((_PAD_M // 128, 128), lambda i: (0, 0)),
            pl.BlockSpec((_PAD_M // 128, 128), lambda i: (0, 0)),
        ],
        out_specs=pl.BlockSpec((1, 1), lambda i: (0, 0)),
        out_shape=jax.ShapeDtypeStruct((1, 1), jnp.float32),
        scratch_shapes=[pltpu.SMEM((1,), jnp.float32)],
        compiler_params=pltpu.CompilerParams(
            dimension_semantics=("arbitrary",)),
    )(out2, tgt2, pos2, mk2)

    return res.reshape(())
